# baseline (speedup 1.0000x reference)
import functools

import jax
import jax.numpy as jnp
import numpy as np
from jax import lax
from jax.experimental import pallas as pl
from jax.experimental.pallas import tpu as pltpu
from jax.experimental.pallas import tpu_sc as plsc

N_NODES = 10000
NPAD = 10240
D_IN = 128
D_HID = 128
D_OUT = 64
N_EDGES = 320000
NW = 32
CH = 128
EPW = ((N_EDGES + NW * 2 * CH - 1) // (NW * 2 * CH)) * 2 * CH
NCH = EPW // CH
EPAD = NW * EPW
ROWS_PER_TILE = NPAD // 16

_mesh = plsc.VectorSubcoreMesh(core_axis_name="c", subcore_axis_name="s")

_PAD_E = EPAD - N_EDGES
_PAD_SRC = np.arange(_PAD_E, dtype=np.int32) % NPAD
_PAD_DST = (N_NODES
            + (np.arange(_PAD_E, dtype=np.int32) % (NPAD - N_NODES)))


@functools.partial(
    pl.kernel,
    out_type=jax.ShapeDtypeStruct((2 * NPAD,), jnp.float32),
    mesh=_mesh,
    scratch_types=[
        pltpu.VMEM((NCH, CH), jnp.int32),
        pltpu.VMEM((CH,), jnp.float32),
        pltpu.VMEM_SHARED((NPAD,), jnp.float32),
    ],
)
def _sc_count(dst_hbm, zeros1_hbm, out_hbm, dst_all, ones_v, acc):
    c = lax.axis_index("c")
    s = lax.axis_index("s")
    wid = s * 2 + c
    for j in range(CH // 16):
        ones_v[pl.ds(j * 16, 16)] = jnp.ones((16,), jnp.float32)
    rbase = s * ROWS_PER_TILE
    pltpu.sync_copy(dst_hbm.at[pl.ds(wid * NCH, NCH)], dst_all)
    pltpu.sync_copy(zeros1_hbm, acc.at[pl.ds(rbase, ROWS_PER_TILE)])
    plsc.subcore_barrier()

    def body(i, carry):
        pltpu.sync_copy(ones_v, acc.at[dst_all.at[i]], add=True)
        return carry

    lax.fori_loop(0, NCH, body, 0)
    plsc.subcore_barrier()
    pltpu.sync_copy(acc.at[pl.ds(rbase, ROWS_PER_TILE)],
                    out_hbm.at[pl.ds(c * NPAD + rbase, ROWS_PER_TILE)])


def _make_sc_scatter(D):
    params = (pltpu.CompilerParams(use_tc_tiling_on_sc=False)
              if D % 128 != 0 else None)

    npass = 2 if D > 64 else 1
    nstg = NCH // npass

    @functools.partial(
        pl.kernel,
        out_type=jax.ShapeDtypeStruct((2 * NPAD, D), jnp.float32),
        mesh=_mesh,
        compiler_params=params,
        scratch_types=[
            pltpu.VMEM((nstg, CH), jnp.int32),
            pltpu.VMEM((nstg, CH), jnp.int32),
            pltpu.VMEM((CH, D), jnp.float32),
            pltpu.VMEM((CH, D), jnp.float32),
            pltpu.VMEM_SHARED((NPAD, D), jnp.float32),
            pltpu.SemaphoreType.DMA,
            pltpu.SemaphoreType.DMA,
        ],
    )
    def k(hs_hbm, src_hbm, dst_hbm, zeros_hbm, out_hbm,
          src_all, dst_all, buf0, buf1, acc, sem0, sem1):
        c = lax.axis_index("c")
        s = lax.axis_index("s")
        wid = s * 2 + c
        rbase = s * ROWS_PER_TILE
        pltpu.sync_copy(zeros_hbm, acc.at[pl.ds(rbase, ROWS_PER_TILE)])
        plsc.subcore_barrier()

        for half in range(npass):
            pltpu.sync_copy(
                src_hbm.at[pl.ds(wid * NCH + half * nstg, nstg)], src_all)
            pltpu.sync_copy(
                dst_hbm.at[pl.ds(wid * NCH + half * nstg, nstg)], dst_all)
            pltpu.async_copy(hs_hbm.at[src_all.at[0]], buf0, sem0)
            pltpu.async_copy(hs_hbm.at[src_all.at[1]], buf1, sem1)

            def body(j, carry):
                a = 2 * j
                b = a + 1
                pltpu.make_async_copy(
                    hs_hbm.at[src_all.at[a]], buf0, sem0).wait()
                pltpu.sync_copy(buf0, acc.at[dst_all.at[a]], add=True)

                @pl.when(a + 2 < nstg)
                def _():
                    pltpu.async_copy(hs_hbm.at[src_all.at[a + 2]], buf0, sem0)

                pltpu.make_async_copy(
                    hs_hbm.at[src_all.at[b]], buf1, sem1).wait()
                pltpu.sync_copy(buf1, acc.at[dst_all.at[b]], add=True)

                @pl.when(b + 2 < nstg)
                def _():
                    pltpu.async_copy(hs_hbm.at[src_all.at[b + 2]], buf1, sem1)

                return carry

            lax.fori_loop(0, nstg // 2, body, 0)
        plsc.subcore_barrier()
        pltpu.sync_copy(acc.at[pl.ds(rbase, ROWS_PER_TILE)],
                        out_hbm.at[pl.ds(c * NPAD + rbase, ROWS_PER_TILE)])

    return k


_sc_scatter_hid = _make_sc_scatter(D_HID)
_sc_scatter_out = _make_sc_scatter(D_OUT)


BN = 512


def _tc1_body(cnt0_ref, cnt1_ref, x_ref, w_ref, dis_ref, hs_ref):
    cnt = cnt0_ref[...] + cnt1_ref[...]
    dis = lax.rsqrt(cnt + 1.0)
    dis_ref[...] = dis
    h = jnp.dot(x_ref[...], w_ref[...], preferred_element_type=jnp.float32)
    hs_ref[...] = h * dis


def _tc1(cnt2, x_p, W1):
    grid = NPAD // BN
    nb = NPAD // BN
    return pl.pallas_call(
        _tc1_body,
        grid=(grid,),
        in_specs=[
            pl.BlockSpec((BN, 1), lambda i: (i, 0)),
            pl.BlockSpec((BN, 1), lambda i: (i + nb, 0)),
            pl.BlockSpec((BN, D_IN), lambda i: (i, 0)),
            pl.BlockSpec((D_IN, D_HID), lambda i: (0, 0)),
        ],
        out_specs=[
            pl.BlockSpec((BN, 1), lambda i: (i, 0)),
            pl.BlockSpec((BN, D_HID), lambda i: (i, 0)),
        ],
        out_shape=[
            jax.ShapeDtypeStruct((NPAD, 1), jnp.float32),
            jax.ShapeDtypeStruct((NPAD, D_HID), jnp.float32),
        ],
    )(cnt2, cnt2, x_p, W1)


def _tc2_body(p0_ref, p1_ref, hs_ref, dis_ref, b_ref, w_ref, hs2_ref):
    agg = p0_ref[...] + p1_ref[...] + hs_ref[...]
    dis = dis_ref[...]
    h1 = jnp.maximum(agg * dis + b_ref[...], 0.0)
    hs2_ref[...] = jnp.dot(h1, w_ref[...],
                           preferred_element_type=jnp.float32) * dis


def _tc2(p, hs1, dis, b1r, W2):
    grid = NPAD // BN
    nb = NPAD // BN
    return pl.pallas_call(
        _tc2_body,
        grid=(grid,),
        in_specs=[
            pl.BlockSpec((BN, D_HID), lambda i: (i, 0)),
            pl.BlockSpec((BN, D_HID), lambda i: (i + nb, 0)),
            pl.BlockSpec((BN, D_HID), lambda i: (i, 0)),
            pl.BlockSpec((BN, 1), lambda i: (i, 0)),
            pl.BlockSpec((1, D_HID), lambda i: (0, 0)),
            pl.BlockSpec((D_HID, D_OUT), lambda i: (0, 0)),
        ],
        out_specs=pl.BlockSpec((BN, D_OUT), lambda i: (i, 0)),
        out_shape=jax.ShapeDtypeStruct((NPAD, D_OUT), jnp.float32),
    )(p, p, hs1, dis, b1r, W2)


def _tc3_body(q0_ref, q1_ref, hs2_ref, dis_ref, b_ref, out_ref):
    agg = q0_ref[...] + q1_ref[...] + hs2_ref[...]
    out_ref[...] = agg * dis_ref[...] + b_ref[...]


def _tc3(q, hs2, dis, b2r):
    grid = NPAD // BN
    nb = NPAD // BN
    return pl.pallas_call(
        _tc3_body,
        grid=(grid,),
        in_specs=[
            pl.BlockSpec((BN, D_OUT), lambda i: (i, 0)),
            pl.BlockSpec((BN, D_OUT), lambda i: (i + nb, 0)),
            pl.BlockSpec((BN, D_OUT), lambda i: (i, 0)),
            pl.BlockSpec((BN, 1), lambda i: (i, 0)),
            pl.BlockSpec((1, D_OUT), lambda i: (0, 0)),
        ],
        out_specs=pl.BlockSpec((BN, D_OUT), lambda i: (i, 0)),
        out_shape=jax.ShapeDtypeStruct((N_NODES, D_OUT), jnp.float32),
    )(q, q, hs2, dis, b2r)


def kernel(x, edge_index, W1, b1, W2, b2):
    src = edge_index[0]
    dst = edge_index[1]
    src_p = jnp.concatenate([src, _PAD_SRC])
    dst_p = jnp.concatenate([dst, _PAD_DST])
    src2 = src_p.reshape(EPAD // CH, CH)
    dst2 = dst_p.reshape(EPAD // CH, CH)
    x_p = jnp.concatenate(
        [x, jnp.zeros((NPAD - N_NODES, D_IN), jnp.float32)], axis=0)
    zeros1 = jnp.zeros((ROWS_PER_TILE,), jnp.float32)
    zeros_hid = jnp.zeros((ROWS_PER_TILE, D_HID), jnp.float32)
    zeros_out = jnp.zeros((ROWS_PER_TILE, D_OUT), jnp.float32)

    cnt = _sc_count(dst2, zeros1)
    dis, hs1 = _tc1(cnt.reshape(2 * NPAD, 1), x_p, W1)
    p = _sc_scatter_hid(hs1, src2, dst2, zeros_hid)
    hs2 = _tc2(p, hs1, dis, b1.reshape(1, D_HID), W2)
    q = _sc_scatter_out(hs2, src2, dst2, zeros_out)
    return _tc3(q, hs2, dis, b2.reshape(1, D_OUT))


# speedup vs baseline: 1.0281x; 1.0281x over previous
import functools

import jax
import jax.numpy as jnp
import numpy as np
from jax import lax
from jax.experimental import pallas as pl
from jax.experimental.pallas import tpu as pltpu
from jax.experimental.pallas import tpu_sc as plsc

N_NODES = 10000
NPAD = 10240
D_IN = 128
D_HID = 128
D_OUT = 64
N_EDGES = 320000
NW = 32
CH = 128
EPW = ((N_EDGES + NW * 2 * CH - 1) // (NW * 2 * CH)) * 2 * CH
NCH = EPW // CH
EPAD = NW * EPW
ROWS_PER_TILE = NPAD // 16

_mesh = plsc.VectorSubcoreMesh(core_axis_name="c", subcore_axis_name="s")

_PAD_E = EPAD - N_EDGES
_PAD_SRC = np.arange(_PAD_E, dtype=np.int32) % NPAD
_PAD_DST = (N_NODES
            + (np.arange(_PAD_E, dtype=np.int32) % (NPAD - N_NODES)))


@functools.partial(
    pl.kernel,
    out_type=jax.ShapeDtypeStruct((2 * NPAD,), jnp.float32),
    mesh=_mesh,
    scratch_types=[
        pltpu.VMEM((NCH, CH), jnp.int32),
        pltpu.VMEM((CH,), jnp.float32),
        pltpu.VMEM_SHARED((NPAD,), jnp.float32),
    ],
)
def _sc_count(dst_hbm, zeros1_hbm, out_hbm, dst_all, ones_v, acc):
    c = lax.axis_index("c")
    s = lax.axis_index("s")
    wid = s * 2 + c
    for j in range(CH // 16):
        ones_v[pl.ds(j * 16, 16)] = jnp.ones((16,), jnp.float32)
    rbase = s * ROWS_PER_TILE
    pltpu.sync_copy(dst_hbm.at[pl.ds(wid * NCH, NCH)], dst_all)
    pltpu.sync_copy(zeros1_hbm, acc.at[pl.ds(rbase, ROWS_PER_TILE)])
    plsc.subcore_barrier()

    def body(i, carry):
        pltpu.sync_copy(ones_v, acc.at[dst_all.at[i]], add=True)
        return carry

    lax.fori_loop(0, NCH, body, 0)
    plsc.subcore_barrier()
    pltpu.sync_copy(acc.at[pl.ds(rbase, ROWS_PER_TILE)],
                    out_hbm.at[pl.ds(c * NPAD + rbase, ROWS_PER_TILE)])


def _make_sc_scatter(D):
    params = (pltpu.CompilerParams(use_tc_tiling_on_sc=False)
              if D % 128 != 0 else None)

    npass = 2 if D > 64 else 1
    nstg = NCH // npass

    @functools.partial(
        pl.kernel,
        out_type=jax.ShapeDtypeStruct((2 * NPAD, D), jnp.float32),
        mesh=_mesh,
        compiler_params=params,
        scratch_types=[
            pltpu.VMEM((nstg, CH), jnp.int32),
            pltpu.VMEM((nstg, CH), jnp.int32),
            pltpu.VMEM((CH, D), jnp.float32),
            pltpu.VMEM((CH, D), jnp.float32),
            pltpu.VMEM_SHARED((NPAD, D), jnp.float32),
            pltpu.SemaphoreType.DMA,
            pltpu.SemaphoreType.DMA,
        ],
    )
    def k(hs_hbm, src_hbm, dst_hbm, zeros_hbm, out_hbm,
          src_all, dst_all, buf0, buf1, acc, sem0, sem1):
        c = lax.axis_index("c")
        s = lax.axis_index("s")
        wid = s * 2 + c
        rbase = s * ROWS_PER_TILE

        @pl.when(c == 0)
        def _():
            pltpu.sync_copy(hs_hbm.at[pl.ds(rbase, ROWS_PER_TILE)],
                            acc.at[pl.ds(rbase, ROWS_PER_TILE)])

        @pl.when(c != 0)
        def _():
            pltpu.sync_copy(zeros_hbm, acc.at[pl.ds(rbase, ROWS_PER_TILE)])

        plsc.subcore_barrier()

        for half in range(npass):
            pltpu.sync_copy(
                src_hbm.at[pl.ds(wid * NCH + half * nstg, nstg)], src_all)
            pltpu.sync_copy(
                dst_hbm.at[pl.ds(wid * NCH + half * nstg, nstg)], dst_all)
            pltpu.async_copy(hs_hbm.at[src_all.at[0]], buf0, sem0)
            pltpu.async_copy(hs_hbm.at[src_all.at[1]], buf1, sem1)

            def body(j, carry):
                a = 2 * j
                b = a + 1
                pltpu.make_async_copy(
                    hs_hbm.at[src_all.at[a]], buf0, sem0).wait()
                pltpu.sync_copy(buf0, acc.at[dst_all.at[a]], add=True)

                @pl.when(a + 2 < nstg)
                def _():
                    pltpu.async_copy(hs_hbm.at[src_all.at[a + 2]], buf0, sem0)

                pltpu.make_async_copy(
                    hs_hbm.at[src_all.at[b]], buf1, sem1).wait()
                pltpu.sync_copy(buf1, acc.at[dst_all.at[b]], add=True)

                @pl.when(b + 2 < nstg)
                def _():
                    pltpu.async_copy(hs_hbm.at[src_all.at[b + 2]], buf1, sem1)

                return carry

            lax.fori_loop(0, nstg // 2, body, 0)
        plsc.subcore_barrier()
        pltpu.sync_copy(acc.at[pl.ds(rbase, ROWS_PER_TILE)],
                        out_hbm.at[pl.ds(c * NPAD + rbase, ROWS_PER_TILE)])

    return k


_sc_scatter_hid = _make_sc_scatter(D_HID)
_sc_scatter_out = _make_sc_scatter(D_OUT)


BN = 512


def _tc1_body(cnt_ref, x_ref, w_ref, dis_ref, hs_ref):
    cnt = cnt_ref[:, 0:1] + cnt_ref[:, 1:2]
    dis = lax.rsqrt(cnt + 1.0)
    dis_ref[...] = dis
    h = jnp.dot(x_ref[...], w_ref[...], preferred_element_type=jnp.float32)
    hs_ref[...] = h * dis


def _tc1(cnt2, x_p, W1):
    grid = NPAD // BN
    return pl.pallas_call(
        _tc1_body,
        grid=(grid,),
        in_specs=[
            pl.BlockSpec((BN, 2), lambda i: (i, 0)),
            pl.BlockSpec((BN, D_IN), lambda i: (i, 0)),
            pl.BlockSpec((D_IN, D_HID), lambda i: (0, 0)),
        ],
        out_specs=[
            pl.BlockSpec((BN, 1), lambda i: (i, 0)),
            pl.BlockSpec((BN, D_HID), lambda i: (i, 0)),
        ],
        out_shape=[
            jax.ShapeDtypeStruct((NPAD, 1), jnp.float32),
            jax.ShapeDtypeStruct((NPAD, D_HID), jnp.float32),
        ],
    )(cnt2, x_p, W1)


def _tc2_body(p0_ref, p1_ref, dis_ref, b_ref, w_ref, hs2_ref):
    agg = p0_ref[...] + p1_ref[...]
    dis = dis_ref[...]
    h1 = jnp.maximum(agg * dis + b_ref[...], 0.0)
    hs2_ref[...] = jnp.dot(h1, w_ref[...],
                           preferred_element_type=jnp.float32) * dis


def _tc2(p, dis, b1r, W2):
    grid = NPAD // BN
    nb = NPAD // BN
    return pl.pallas_call(
        _tc2_body,
        grid=(grid,),
        in_specs=[
            pl.BlockSpec((BN, D_HID), lambda i: (i, 0)),
            pl.BlockSpec((BN, D_HID), lambda i: (i + nb, 0)),
            pl.BlockSpec((BN, 1), lambda i: (i, 0)),
            pl.BlockSpec((1, D_HID), lambda i: (0, 0)),
            pl.BlockSpec((D_HID, D_OUT), lambda i: (0, 0)),
        ],
        out_specs=pl.BlockSpec((BN, D_OUT), lambda i: (i, 0)),
        out_shape=jax.ShapeDtypeStruct((NPAD, D_OUT), jnp.float32),
    )(p, p, dis, b1r, W2)


def _tc3_body(q0_ref, q1_ref, dis_ref, b_ref, out_ref):
    agg = q0_ref[...] + q1_ref[...]
    out_ref[...] = agg * dis_ref[...] + b_ref[...]


def _tc3(q, dis, b2r):
    grid = NPAD // BN
    nb = NPAD // BN
    return pl.pallas_call(
        _tc3_body,
        grid=(grid,),
        in_specs=[
            pl.BlockSpec((BN, D_OUT), lambda i: (i, 0)),
            pl.BlockSpec((BN, D_OUT), lambda i: (i + nb, 0)),
            pl.BlockSpec((BN, 1), lambda i: (i, 0)),
            pl.BlockSpec((1, D_OUT), lambda i: (0, 0)),
        ],
        out_specs=pl.BlockSpec((BN, D_OUT), lambda i: (i, 0)),
        out_shape=jax.ShapeDtypeStruct((N_NODES, D_OUT), jnp.float32),
    )(q, q, dis, b2r)


def kernel(x, edge_index, W1, b1, W2, b2):
    src = edge_index[0]
    dst = edge_index[1]
    src_p = jnp.concatenate([src, _PAD_SRC])
    dst_p = jnp.concatenate([dst, _PAD_DST])
    src2 = src_p.reshape(EPAD // CH, CH)
    dst2 = dst_p.reshape(EPAD // CH, CH)
    x_p = jnp.concatenate(
        [x, jnp.zeros((NPAD - N_NODES, D_IN), jnp.float32)], axis=0)
    zeros1 = jnp.zeros((ROWS_PER_TILE,), jnp.float32)
    zeros_hid = jnp.zeros((ROWS_PER_TILE, D_HID), jnp.float32)
    zeros_out = jnp.zeros((ROWS_PER_TILE, D_OUT), jnp.float32)

    cnt = _sc_count(dst2, zeros1)
    cnt2 = cnt.reshape(2, NPAD).T
    dis, hs1 = _tc1(cnt2, x_p, W1)
    p = _sc_scatter_hid(hs1, src2, dst2, zeros_hid)
    hs2 = _tc2(p, dis, b1.reshape(1, D_HID), W2)
    q = _sc_scatter_out(hs2, src2, dst2, zeros_out)
    return _tc3(q, dis, b2.reshape(1, D_OUT))


# speedup vs baseline: 1.0769x; 1.0475x over previous
import functools

import jax
import jax.numpy as jnp
import numpy as np
from jax import lax
from jax.experimental import pallas as pl
from jax.experimental.pallas import tpu as pltpu
from jax.experimental.pallas import tpu_sc as plsc

N_NODES = 10000
NPAD = 10240
D_IN = 128
D_HID = 128
D_OUT = 64
N_EDGES = 320000
NW = 32
CH = 128
EPW = ((N_EDGES + NW * 2 * CH - 1) // (NW * 2 * CH)) * 2 * CH
NCH = EPW // CH
EPAD = NW * EPW
ROWS_PER_TILE = NPAD // 16

_mesh = plsc.VectorSubcoreMesh(core_axis_name="c", subcore_axis_name="s")

_PAD_E = EPAD - N_EDGES
_PAD_SRC = np.arange(_PAD_E, dtype=np.int32) % NPAD
_PAD_DST = (N_NODES
            + (np.arange(_PAD_E, dtype=np.int32) % (NPAD - N_NODES)))


@functools.partial(
    pl.kernel,
    out_type=jax.ShapeDtypeStruct((2 * NPAD,), jnp.float32),
    mesh=_mesh,
    scratch_types=[
        pltpu.VMEM((NCH, CH), jnp.int32),
        pltpu.VMEM((CH,), jnp.float32),
        pltpu.VMEM_SHARED((NPAD,), jnp.float32),
    ],
)
def _sc_count(dst_hbm, zeros1_hbm, out_hbm, dst_all, ones_v, acc):
    c = lax.axis_index("c")
    s = lax.axis_index("s")
    wid = s * 2 + c
    for j in range(CH // 16):
        ones_v[pl.ds(j * 16, 16)] = jnp.ones((16,), jnp.float32)
    rbase = s * ROWS_PER_TILE
    pltpu.sync_copy(dst_hbm.at[pl.ds(wid * NCH, NCH)], dst_all)
    pltpu.sync_copy(zeros1_hbm, acc.at[pl.ds(rbase, ROWS_PER_TILE)])
    plsc.subcore_barrier()

    def body(i, carry):
        pltpu.sync_copy(ones_v, acc.at[dst_all.at[i]], add=True)
        return carry

    lax.fori_loop(0, NCH, body, 0)
    plsc.subcore_barrier()
    pltpu.sync_copy(acc.at[pl.ds(rbase, ROWS_PER_TILE)],
                    out_hbm.at[pl.ds(c * NPAD + rbase, ROWS_PER_TILE)])


def _make_sc_scatter(D):
    params = (pltpu.CompilerParams(use_tc_tiling_on_sc=False)
              if D % 128 != 0 else None)

    npass = 2 if D > 64 else 1
    nstg = NCH // npass

    @functools.partial(
        pl.kernel,
        out_type=jax.ShapeDtypeStruct((2 * NPAD, D), jnp.float32),
        mesh=_mesh,
        compiler_params=params,
        scratch_types=[
            pltpu.VMEM((nstg, CH), jnp.int32),
            pltpu.VMEM((nstg, CH), jnp.int32),
            pltpu.VMEM((CH, D), jnp.float32),
            pltpu.VMEM((CH, D), jnp.float32),
            pltpu.VMEM_SHARED((NPAD, D), jnp.float32),
            pltpu.SemaphoreType.DMA,
            pltpu.SemaphoreType.DMA,
        ],
    )
    def k(hs_hbm, src_hbm, dst_hbm, zeros_hbm, out_hbm,
          src_all, dst_all, buf0, buf1, acc, sem0, sem1):
        c = lax.axis_index("c")
        s = lax.axis_index("s")
        wid = s * 2 + c
        rbase = s * ROWS_PER_TILE

        @pl.when(c == 0)
        def _():
            pltpu.sync_copy(hs_hbm.at[pl.ds(rbase, ROWS_PER_TILE)],
                            acc.at[pl.ds(rbase, ROWS_PER_TILE)])

        @pl.when(c != 0)
        def _():
            pltpu.sync_copy(zeros_hbm, acc.at[pl.ds(rbase, ROWS_PER_TILE)])

        plsc.subcore_barrier()

        for half in range(npass):
            pltpu.sync_copy(
                src_hbm.at[pl.ds(wid * NCH + half * nstg, nstg)], src_all)
            pltpu.sync_copy(
                dst_hbm.at[pl.ds(wid * NCH + half * nstg, nstg)], dst_all)
            pltpu.async_copy(hs_hbm.at[src_all.at[0]], buf0, sem0)
            pltpu.async_copy(hs_hbm.at[src_all.at[1]], buf1, sem1)

            def body(j, carry):
                a = 2 * j
                b = a + 1
                pltpu.make_async_copy(
                    hs_hbm.at[src_all.at[a]], buf0, sem0).wait()
                pltpu.sync_copy(buf0, acc.at[dst_all.at[a]], add=True)

                @pl.when(a + 2 < nstg)
                def _():
                    pltpu.async_copy(hs_hbm.at[src_all.at[a + 2]], buf0, sem0)

                pltpu.make_async_copy(
                    hs_hbm.at[src_all.at[b]], buf1, sem1).wait()
                pltpu.sync_copy(buf1, acc.at[dst_all.at[b]], add=True)

                @pl.when(b + 2 < nstg)
                def _():
                    pltpu.async_copy(hs_hbm.at[src_all.at[b + 2]], buf1, sem1)

                return carry

            lax.fori_loop(0, nstg // 2, body, 0)
        plsc.subcore_barrier()
        pltpu.sync_copy(acc.at[pl.ds(rbase, ROWS_PER_TILE)],
                        out_hbm.at[pl.ds(c * NPAD + rbase, ROWS_PER_TILE)])

    return k


_sc_scatter_hid = _make_sc_scatter(D_HID)
_sc_scatter_out = _make_sc_scatter(D_OUT)


BN = 1024


def _tc1_body(cnt_ref, x_ref, w_ref, dis_ref, hs_ref):
    cnt = cnt_ref[:, 0:1] + cnt_ref[:, 1:2]
    dis = lax.rsqrt(cnt + 1.0)
    dis_ref[...] = dis
    h = jnp.dot(x_ref[...], w_ref[...], preferred_element_type=jnp.float32)
    hs_ref[...] = h * dis


def _tc1(cnt2, x_p, W1):
    grid = NPAD // BN
    return pl.pallas_call(
        _tc1_body,
        grid=(grid,),
        in_specs=[
            pl.BlockSpec((BN, 2), lambda i: (i, 0)),
            pl.BlockSpec((BN, D_IN), lambda i: (i, 0)),
            pl.BlockSpec((D_IN, D_HID), lambda i: (0, 0)),
        ],
        out_specs=[
            pl.BlockSpec((BN, 1), lambda i: (i, 0)),
            pl.BlockSpec((BN, D_HID), lambda i: (i, 0)),
        ],
        out_shape=[
            jax.ShapeDtypeStruct((NPAD, 1), jnp.float32),
            jax.ShapeDtypeStruct((NPAD, D_HID), jnp.float32),
        ],
    )(cnt2, x_p, W1)


def _tc2_body(p0_ref, p1_ref, dis_ref, b_ref, w_ref, hs2_ref):
    agg = p0_ref[...] + p1_ref[...]
    dis = dis_ref[...]
    h1 = jnp.maximum(agg * dis + b_ref[...], 0.0)
    hs2_ref[...] = jnp.dot(h1, w_ref[...],
                           preferred_element_type=jnp.float32) * dis


def _tc2(p, dis, b1r, W2):
    grid = NPAD // BN
    nb = NPAD // BN
    return pl.pallas_call(
        _tc2_body,
        grid=(grid,),
        in_specs=[
            pl.BlockSpec((BN, D_HID), lambda i: (i, 0)),
            pl.BlockSpec((BN, D_HID), lambda i: (i + nb, 0)),
            pl.BlockSpec((BN, 1), lambda i: (i, 0)),
            pl.BlockSpec((1, D_HID), lambda i: (0, 0)),
            pl.BlockSpec((D_HID, D_OUT), lambda i: (0, 0)),
        ],
        out_specs=pl.BlockSpec((BN, D_OUT), lambda i: (i, 0)),
        out_shape=jax.ShapeDtypeStruct((NPAD, D_OUT), jnp.float32),
    )(p, p, dis, b1r, W2)


def _tc3_body(q0_ref, q1_ref, dis_ref, b_ref, out_ref):
    agg = q0_ref[...] + q1_ref[...]
    out_ref[...] = agg * dis_ref[...] + b_ref[...]


def _tc3(q, dis, b2r):
    grid = NPAD // BN
    nb = NPAD // BN
    return pl.pallas_call(
        _tc3_body,
        grid=(grid,),
        in_specs=[
            pl.BlockSpec((BN, D_OUT), lambda i: (i, 0)),
            pl.BlockSpec((BN, D_OUT), lambda i: (i + nb, 0)),
            pl.BlockSpec((BN, 1), lambda i: (i, 0)),
            pl.BlockSpec((1, D_OUT), lambda i: (0, 0)),
        ],
        out_specs=pl.BlockSpec((BN, D_OUT), lambda i: (i, 0)),
        out_shape=jax.ShapeDtypeStruct((N_NODES, D_OUT), jnp.float32),
    )(q, q, dis, b2r)


def kernel(x, edge_index, W1, b1, W2, b2):
    src = edge_index[0]
    dst = edge_index[1]
    src_p = jnp.concatenate([src, _PAD_SRC])
    dst_p = jnp.concatenate([dst, _PAD_DST])
    src2 = src_p.reshape(EPAD // CH, CH)
    dst2 = dst_p.reshape(EPAD // CH, CH)
    x_p = jnp.concatenate(
        [x, jnp.zeros((NPAD - N_NODES, D_IN), jnp.float32)], axis=0)
    zeros1 = jnp.zeros((ROWS_PER_TILE,), jnp.float32)
    zeros_hid = jnp.zeros((ROWS_PER_TILE, D_HID), jnp.float32)
    zeros_out = jnp.zeros((ROWS_PER_TILE, D_OUT), jnp.float32)

    cnt = _sc_count(dst2, zeros1)
    cnt2 = cnt.reshape(2, NPAD).T
    dis, hs1 = _tc1(cnt2, x_p, W1)
    p = _sc_scatter_hid(hs1, src2, dst2, zeros_hid)
    hs2 = _tc2(p, dis, b1.reshape(1, D_HID), W2)
    q = _sc_scatter_out(hs2, src2, dst2, zeros_out)
    return _tc3(q, dis, b2.reshape(1, D_OUT))


# speedup vs baseline: 1.1029x; 1.0241x over previous
import functools

import jax
import jax.numpy as jnp
import numpy as np
from jax import lax
from jax.experimental import pallas as pl
from jax.experimental.pallas import tpu as pltpu
from jax.experimental.pallas import tpu_sc as plsc

N_NODES = 10000
NPAD = 10240
D_IN = 128
D_HID = 128
D_OUT = 64
N_EDGES = 320000
NW = 32
CH = 128
EPW = ((N_EDGES + NW * 2 * CH - 1) // (NW * 2 * CH)) * 2 * CH
NCH = EPW // CH
EPAD = NW * EPW
ROWS_PER_TILE = NPAD // 16

_mesh = plsc.VectorSubcoreMesh(core_axis_name="c", subcore_axis_name="s")

_PAD_E = EPAD - N_EDGES
_PAD_SRC = np.arange(_PAD_E, dtype=np.int32) % NPAD
_PAD_DST = (N_NODES
            + (np.arange(_PAD_E, dtype=np.int32) % (NPAD - N_NODES)))


@functools.partial(
    pl.kernel,
    out_type=jax.ShapeDtypeStruct((2 * NPAD,), jnp.float32),
    mesh=_mesh,
    scratch_types=[
        pltpu.VMEM((NCH, CH), jnp.int32),
        pltpu.VMEM((CH,), jnp.float32),
        pltpu.VMEM_SHARED((NPAD,), jnp.float32),
    ],
)
def _sc_count(dst_hbm, zeros1_hbm, out_hbm, dst_all, ones_v, acc):
    c = lax.axis_index("c")
    s = lax.axis_index("s")
    wid = s * 2 + c
    for j in range(CH // 16):
        ones_v[pl.ds(j * 16, 16)] = jnp.ones((16,), jnp.float32)
    rbase = s * ROWS_PER_TILE
    pltpu.sync_copy(dst_hbm.at[pl.ds(wid * NCH, NCH)], dst_all)
    pltpu.sync_copy(zeros1_hbm, acc.at[pl.ds(rbase, ROWS_PER_TILE)])
    plsc.subcore_barrier()

    def body(i, carry):
        pltpu.sync_copy(ones_v, acc.at[dst_all.at[i]], add=True)
        return carry

    lax.fori_loop(0, NCH, body, 0)
    plsc.subcore_barrier()
    pltpu.sync_copy(acc.at[pl.ds(rbase, ROWS_PER_TILE)],
                    out_hbm.at[pl.ds(c * NPAD + rbase, ROWS_PER_TILE)])


def _make_sc_scatter(D):
    params = (pltpu.CompilerParams(use_tc_tiling_on_sc=False)
              if D % 128 != 0 else None)

    npass = 2 if D > 64 else 1
    nstg = NCH // npass

    @functools.partial(
        pl.kernel,
        out_type=jax.ShapeDtypeStruct((2 * NPAD, D), jnp.float32),
        mesh=_mesh,
        compiler_params=params,
        scratch_types=[
            pltpu.VMEM((nstg, CH), jnp.int32),
            pltpu.VMEM((nstg, CH), jnp.int32),
            pltpu.VMEM((CH, D), jnp.float32),
            pltpu.VMEM((CH, D), jnp.float32),
            pltpu.VMEM_SHARED((NPAD, D), jnp.float32),
            pltpu.SemaphoreType.DMA,
            pltpu.SemaphoreType.DMA,
        ],
    )
    def k(hs_hbm, src_hbm, dst_hbm, zeros_hbm, out_hbm,
          src_all, dst_all, buf0, buf1, acc, sem0, sem1):
        c = lax.axis_index("c")
        s = lax.axis_index("s")
        wid = s * 2 + c
        rbase = s * ROWS_PER_TILE

        @pl.when(c == 0)
        def _():
            pltpu.sync_copy(hs_hbm.at[pl.ds(rbase, ROWS_PER_TILE)],
                            acc.at[pl.ds(rbase, ROWS_PER_TILE)])

        @pl.when(c != 0)
        def _():
            pltpu.sync_copy(zeros_hbm, acc.at[pl.ds(rbase, ROWS_PER_TILE)])

        plsc.subcore_barrier()

        for half in range(npass):
            pltpu.sync_copy(
                src_hbm.at[pl.ds(wid * NCH + half * nstg, nstg)], src_all)
            pltpu.sync_copy(
                dst_hbm.at[pl.ds(wid * NCH + half * nstg, nstg)], dst_all)
            pltpu.async_copy(hs_hbm.at[src_all.at[0]], buf0, sem0)
            pltpu.async_copy(hs_hbm.at[src_all.at[1]], buf1, sem1)

            def body(j, carry):
                a = 2 * j
                b = a + 1
                pltpu.make_async_copy(
                    hs_hbm.at[src_all.at[a]], buf0, sem0).wait()
                pltpu.sync_copy(buf0, acc.at[dst_all.at[a]], add=True)

                @pl.when(a + 2 < nstg)
                def _():
                    pltpu.async_copy(hs_hbm.at[src_all.at[a + 2]], buf0, sem0)

                pltpu.make_async_copy(
                    hs_hbm.at[src_all.at[b]], buf1, sem1).wait()
                pltpu.sync_copy(buf1, acc.at[dst_all.at[b]], add=True)

                @pl.when(b + 2 < nstg)
                def _():
                    pltpu.async_copy(hs_hbm.at[src_all.at[b + 2]], buf1, sem1)

                return carry

            lax.fori_loop(0, nstg // 2, body, 0)
        plsc.subcore_barrier()
        pltpu.sync_copy(acc.at[pl.ds(rbase, ROWS_PER_TILE)],
                        out_hbm.at[pl.ds(c * NPAD + rbase, ROWS_PER_TILE)])

    return k


_sc_scatter_hid = _make_sc_scatter(D_HID)
_sc_scatter_out = _make_sc_scatter(D_OUT)


BN = 2048


def _tc1_body(cnt_ref, x_ref, w_ref, dis_ref, hs_ref):
    cnt = cnt_ref[:, 0:1] + cnt_ref[:, 1:2]
    dis = lax.rsqrt(cnt + 1.0)
    dis_ref[...] = dis
    h = jnp.dot(x_ref[...], w_ref[...], preferred_element_type=jnp.float32)
    hs_ref[...] = h * dis


def _tc1(cnt2, x_p, W1):
    grid = NPAD // BN
    return pl.pallas_call(
        _tc1_body,
        grid=(grid,),
        in_specs=[
            pl.BlockSpec((BN, 2), lambda i: (i, 0)),
            pl.BlockSpec((BN, D_IN), lambda i: (i, 0)),
            pl.BlockSpec((D_IN, D_HID), lambda i: (0, 0)),
        ],
        out_specs=[
            pl.BlockSpec((BN, 1), lambda i: (i, 0)),
            pl.BlockSpec((BN, D_HID), lambda i: (i, 0)),
        ],
        out_shape=[
            jax.ShapeDtypeStruct((NPAD, 1), jnp.float32),
            jax.ShapeDtypeStruct((NPAD, D_HID), jnp.float32),
        ],
    )(cnt2, x_p, W1)


def _tc2_body(p0_ref, p1_ref, dis_ref, b_ref, w_ref, hs2_ref):
    agg = p0_ref[...] + p1_ref[...]
    dis = dis_ref[...]
    h1 = jnp.maximum(agg * dis + b_ref[...], 0.0)
    hs2_ref[...] = jnp.dot(h1, w_ref[...],
                           preferred_element_type=jnp.float32) * dis


def _tc2(p, dis, b1r, W2):
    grid = NPAD // BN
    nb = NPAD // BN
    return pl.pallas_call(
        _tc2_body,
        grid=(grid,),
        in_specs=[
            pl.BlockSpec((BN, D_HID), lambda i: (i, 0)),
            pl.BlockSpec((BN, D_HID), lambda i: (i + nb, 0)),
            pl.BlockSpec((BN, 1), lambda i: (i, 0)),
            pl.BlockSpec((1, D_HID), lambda i: (0, 0)),
            pl.BlockSpec((D_HID, D_OUT), lambda i: (0, 0)),
        ],
        out_specs=pl.BlockSpec((BN, D_OUT), lambda i: (i, 0)),
        out_shape=jax.ShapeDtypeStruct((NPAD, D_OUT), jnp.float32),
    )(p, p, dis, b1r, W2)


def _tc3_body(q0_ref, q1_ref, dis_ref, b_ref, out_ref):
    agg = q0_ref[...] + q1_ref[...]
    out_ref[...] = agg * dis_ref[...] + b_ref[...]


def _tc3(q, dis, b2r):
    grid = NPAD // BN
    nb = NPAD // BN
    return pl.pallas_call(
        _tc3_body,
        grid=(grid,),
        in_specs=[
            pl.BlockSpec((BN, D_OUT), lambda i: (i, 0)),
            pl.BlockSpec((BN, D_OUT), lambda i: (i + nb, 0)),
            pl.BlockSpec((BN, 1), lambda i: (i, 0)),
            pl.BlockSpec((1, D_OUT), lambda i: (0, 0)),
        ],
        out_specs=pl.BlockSpec((BN, D_OUT), lambda i: (i, 0)),
        out_shape=jax.ShapeDtypeStruct((N_NODES, D_OUT), jnp.float32),
    )(q, q, dis, b2r)


def kernel(x, edge_index, W1, b1, W2, b2):
    src = edge_index[0]
    dst = edge_index[1]
    src_p = jnp.concatenate([src, _PAD_SRC])
    dst_p = jnp.concatenate([dst, _PAD_DST])
    src2 = src_p.reshape(EPAD // CH, CH)
    dst2 = dst_p.reshape(EPAD // CH, CH)
    x_p = jnp.concatenate(
        [x, jnp.zeros((NPAD - N_NODES, D_IN), jnp.float32)], axis=0)
    zeros1 = jnp.zeros((ROWS_PER_TILE,), jnp.float32)
    zeros_hid = jnp.zeros((ROWS_PER_TILE, D_HID), jnp.float32)
    zeros_out = jnp.zeros((ROWS_PER_TILE, D_OUT), jnp.float32)

    cnt = _sc_count(dst2, zeros1)
    cnt2 = cnt.reshape(2, NPAD).T
    dis, hs1 = _tc1(cnt2, x_p, W1)
    p = _sc_scatter_hid(hs1, src2, dst2, zeros_hid)
    hs2 = _tc2(p, dis, b1.reshape(1, D_HID), W2)
    q = _sc_scatter_out(hs2, src2, dst2, zeros_out)
    return _tc3(q, dis, b2.reshape(1, D_OUT))


# speedup vs baseline: 1.1110x; 1.0073x over previous
import functools

import jax
import jax.numpy as jnp
import numpy as np
from jax import lax
from jax.experimental import pallas as pl
from jax.experimental.pallas import tpu as pltpu
from jax.experimental.pallas import tpu_sc as plsc

N_NODES = 10000
NPAD = 10240
D_IN = 128
D_HID = 128
D_OUT = 64
N_EDGES = 320000
NW = 32
CH = 128
EPW = ((N_EDGES + NW * 2 * CH - 1) // (NW * 2 * CH)) * 2 * CH
NCH = EPW // CH
EPAD = NW * EPW
ROWS_PER_TILE = NPAD // 16

_mesh = plsc.VectorSubcoreMesh(core_axis_name="c", subcore_axis_name="s")

_PAD_E = EPAD - N_EDGES
_PAD_SRC = np.arange(_PAD_E, dtype=np.int32) % NPAD
_PAD_DST = N_NODES + (np.arange(_PAD_E, dtype=np.int32) % (NPAD - N_NODES))


@functools.partial(
    pl.kernel,
    out_type=jax.ShapeDtypeStruct((2 * NPAD,), jnp.float32),
    mesh=_mesh,
    scratch_types=[
        pltpu.VMEM((NCH, CH), jnp.int32),
        pltpu.VMEM((CH,), jnp.float32),
        pltpu.VMEM_SHARED((NPAD,), jnp.float32),
    ],
)
def _sc_count(dst_hbm, zeros1_hbm, out_hbm, dst_all, ones_v, acc):
    c = lax.axis_index("c")
    s = lax.axis_index("s")
    wid = s * 2 + c
    for j in range(CH // 16):
        ones_v[pl.ds(j * 16, 16)] = jnp.ones((16,), jnp.float32)
    rbase = s * ROWS_PER_TILE
    pltpu.sync_copy(dst_hbm.at[pl.ds(wid * NCH, NCH)], dst_all)
    pltpu.sync_copy(zeros1_hbm, acc.at[pl.ds(rbase, ROWS_PER_TILE)])
    plsc.subcore_barrier()

    def body(i, carry):
        pltpu.sync_copy(ones_v, acc.at[dst_all.at[i]], add=True)
        return carry

    lax.fori_loop(0, NCH, body, 0)
    plsc.subcore_barrier()
    pltpu.sync_copy(acc.at[pl.ds(rbase, ROWS_PER_TILE)],
                    out_hbm.at[pl.ds(c * NPAD + rbase, ROWS_PER_TILE)])


def _make_sc_scatter(D):
    params = (pltpu.CompilerParams(use_tc_tiling_on_sc=False)
              if D % 128 != 0 else None)

    npass = 2 if D > 64 else 1
    nstg = NCH // npass

    @functools.partial(
        pl.kernel,
        out_type=jax.ShapeDtypeStruct((2 * NPAD, D), jnp.float32),
        mesh=_mesh,
        compiler_params=params,
        scratch_types=[
            pltpu.VMEM((nstg, CH), jnp.int32),
            pltpu.VMEM((nstg, CH), jnp.int32),
            pltpu.VMEM((CH, D), jnp.float32),
            pltpu.VMEM((CH, D), jnp.float32),
            pltpu.VMEM_SHARED((NPAD, D), jnp.float32),
            pltpu.SemaphoreType.DMA,
            pltpu.SemaphoreType.DMA,
        ],
    )
    def k(hs_hbm, src_hbm, dst_hbm, zeros_hbm,
          out_hbm, src_all, dst_all, buf0, buf1, acc, sem0, sem1):
        c = lax.axis_index("c")
        s = lax.axis_index("s")
        wid = s * 2 + c
        rbase = s * ROWS_PER_TILE

        @pl.when(c == 0)
        def _():
            pltpu.sync_copy(hs_hbm.at[pl.ds(rbase, ROWS_PER_TILE)],
                            acc.at[pl.ds(rbase, ROWS_PER_TILE)])

        @pl.when(c != 0)
        def _():
            pltpu.sync_copy(zeros_hbm, acc.at[pl.ds(rbase, ROWS_PER_TILE)])

        plsc.subcore_barrier()

        for half in range(npass):
            pltpu.sync_copy(
                src_hbm.at[pl.ds(wid * NCH + half * nstg, nstg)], src_all)
            pltpu.sync_copy(
                dst_hbm.at[pl.ds(wid * NCH + half * nstg, nstg)], dst_all)
            pltpu.async_copy(hs_hbm.at[src_all.at[0]], buf0, sem0)
            pltpu.async_copy(hs_hbm.at[src_all.at[1]], buf1, sem1)

            def body(j, carry):
                a = 2 * j
                b = a + 1
                pltpu.make_async_copy(
                    hs_hbm.at[src_all.at[a]], buf0, sem0).wait()
                pltpu.sync_copy(buf0, acc.at[dst_all.at[a]], add=True)

                @pl.when(a + 2 < nstg)
                def _():
                    pltpu.async_copy(hs_hbm.at[src_all.at[a + 2]], buf0, sem0)

                pltpu.make_async_copy(
                    hs_hbm.at[src_all.at[b]], buf1, sem1).wait()
                pltpu.sync_copy(buf1, acc.at[dst_all.at[b]], add=True)

                @pl.when(b + 2 < nstg)
                def _():
                    pltpu.async_copy(hs_hbm.at[src_all.at[b + 2]], buf1, sem1)

                return carry

            lax.fori_loop(0, nstg // 2, body, 0)
        plsc.subcore_barrier()
        pltpu.sync_copy(acc.at[pl.ds(rbase, ROWS_PER_TILE)],
                        out_hbm.at[pl.ds(c * NPAD + rbase, ROWS_PER_TILE)])

    return k


_sc_scatter_hid = _make_sc_scatter(D_HID)
_sc_scatter_out = _make_sc_scatter(D_OUT)


BN = 2048


def _tc1_body(cnt_ref, x_ref, w_ref, dis_ref, hs_ref):
    cnt = cnt_ref[:, 0:1] + cnt_ref[:, 1:2]
    dis = lax.rsqrt(cnt + 1.0)
    dis_ref[...] = dis
    h = jnp.dot(x_ref[...], w_ref[...], preferred_element_type=jnp.float32)
    hs_ref[...] = h * dis


def _tc1(cnt2, x_p, W1):
    grid = NPAD // BN
    return pl.pallas_call(
        _tc1_body,
        grid=(grid,),
        in_specs=[
            pl.BlockSpec((BN, 2), lambda i: (i, 0)),
            pl.BlockSpec((BN, D_IN), lambda i: (i, 0)),
            pl.BlockSpec((D_IN, D_HID), lambda i: (0, 0)),
        ],
        out_specs=[
            pl.BlockSpec((BN, 1), lambda i: (i, 0)),
            pl.BlockSpec((BN, D_HID), lambda i: (i, 0)),
        ],
        out_shape=[
            jax.ShapeDtypeStruct((NPAD, 1), jnp.float32),
            jax.ShapeDtypeStruct((NPAD, D_HID), jnp.float32),
        ],
    )(cnt2, x_p, W1)


def _tc2_body(p0_ref, p1_ref, dis_ref, b_ref, w_ref, hs2_ref):
    agg = p0_ref[...] + p1_ref[...]
    dis = dis_ref[...]
    h1 = jnp.maximum(agg * dis + b_ref[...], 0.0)
    hs2_ref[...] = jnp.dot(h1, w_ref[...],
                           preferred_element_type=jnp.float32) * dis


def _tc2(p, dis, b1r, W2):
    grid = NPAD // BN
    nb = NPAD // BN
    return pl.pallas_call(
        _tc2_body,
        grid=(grid,),
        in_specs=[
            pl.BlockSpec((BN, D_HID), lambda i: (i, 0)),
            pl.BlockSpec((BN, D_HID), lambda i: (i + nb, 0)),
            pl.BlockSpec((BN, 1), lambda i: (i, 0)),
            pl.BlockSpec((1, D_HID), lambda i: (0, 0)),
            pl.BlockSpec((D_HID, D_OUT), lambda i: (0, 0)),
        ],
        out_specs=pl.BlockSpec((BN, D_OUT), lambda i: (i, 0)),
        out_shape=jax.ShapeDtypeStruct((NPAD, D_OUT), jnp.float32),
    )(p, p, dis, b1r, W2)


def _tc3_body(q0_ref, q1_ref, dis_ref, b_ref, out_ref):
    agg = q0_ref[...] + q1_ref[...]
    out_ref[...] = agg * dis_ref[...] + b_ref[...]


def _tc3(q, dis, b2r):
    grid = NPAD // BN
    nb = NPAD // BN
    return pl.pallas_call(
        _tc3_body,
        grid=(grid,),
        in_specs=[
            pl.BlockSpec((BN, D_OUT), lambda i: (i, 0)),
            pl.BlockSpec((BN, D_OUT), lambda i: (i + nb, 0)),
            pl.BlockSpec((BN, 1), lambda i: (i, 0)),
            pl.BlockSpec((1, D_OUT), lambda i: (0, 0)),
        ],
        out_specs=pl.BlockSpec((BN, D_OUT), lambda i: (i, 0)),
        out_shape=jax.ShapeDtypeStruct((N_NODES, D_OUT), jnp.float32),
    )(q, q, dis, b2r)


def kernel(x, edge_index, W1, b1, W2, b2):
    src2 = jnp.concatenate([edge_index[0], _PAD_SRC]).reshape(EPAD // CH, CH)
    dst2 = jnp.concatenate([edge_index[1], _PAD_DST]).reshape(EPAD // CH, CH)
    zeros1 = jnp.zeros((ROWS_PER_TILE,), jnp.float32)
    zeros_hid = jnp.zeros((ROWS_PER_TILE, D_HID), jnp.float32)
    zeros_out = jnp.zeros((ROWS_PER_TILE, D_OUT), jnp.float32)

    cnt = _sc_count(dst2, zeros1)
    cnt2 = cnt.reshape(2, NPAD).T
    dis, hs1 = _tc1(cnt2, x, W1)
    p = _sc_scatter_hid(hs1, src2, dst2, zeros_hid)
    hs2 = _tc2(p, dis, b1.reshape(1, D_HID), W2)
    q = _sc_scatter_out(hs2, src2, dst2, zeros_out)
    return _tc3(q, dis, b2.reshape(1, D_OUT))


# speedup vs baseline: 1.1791x; 1.0614x over previous
import functools

import jax
import jax.numpy as jnp
import numpy as np
from jax import lax
from jax.experimental import pallas as pl
from jax.experimental.pallas import tpu as pltpu
from jax.experimental.pallas import tpu_sc as plsc

N_NODES = 10000
NPAD = 10240
D_IN = 128
D_HID = 128
D_OUT = 64
N_EDGES = 320000
NW = 32
CH = 128
EPW = ((N_EDGES + NW * 2 * CH - 1) // (NW * 2 * CH)) * 2 * CH
NCH = EPW // CH
EPAD = NW * EPW
ROWS_PER_TILE = NPAD // 16

_mesh = plsc.VectorSubcoreMesh(core_axis_name="c", subcore_axis_name="s")

_PAD_E = EPAD - N_EDGES
_PAD_SRC = np.arange(_PAD_E, dtype=np.int32) % NPAD
_PAD_DST = N_NODES + (np.arange(_PAD_E, dtype=np.int32) % (NPAD - N_NODES))


@functools.partial(
    pl.kernel,
    out_type=jax.ShapeDtypeStruct((2 * NPAD,), jnp.float32),
    mesh=_mesh,
    scratch_types=[
        pltpu.VMEM((NCH, CH), jnp.int32),
        pltpu.VMEM((CH,), jnp.float32),
        pltpu.VMEM_SHARED((NPAD,), jnp.float32),
    ],
)
def _sc_count(dst_hbm, zeros1_hbm, out_hbm, dst_all, ones_v, acc):
    c = lax.axis_index("c")
    s = lax.axis_index("s")
    wid = s * 2 + c
    for j in range(CH // 16):
        ones_v[pl.ds(j * 16, 16)] = jnp.ones((16,), jnp.float32)
    rbase = s * ROWS_PER_TILE
    pltpu.sync_copy(dst_hbm.at[pl.ds(wid * NCH, NCH)], dst_all)
    pltpu.sync_copy(zeros1_hbm, acc.at[pl.ds(rbase, ROWS_PER_TILE)])
    plsc.subcore_barrier()

    def body(i, carry):
        pltpu.sync_copy(ones_v, acc.at[dst_all.at[i]], add=True)
        return carry

    lax.fori_loop(0, NCH, body, 0)
    plsc.subcore_barrier()
    pltpu.sync_copy(acc.at[pl.ds(rbase, ROWS_PER_TILE)],
                    out_hbm.at[pl.ds(c * NPAD + rbase, ROWS_PER_TILE)])


def _make_sc_scatter(D):
    params = (pltpu.CompilerParams(use_tc_tiling_on_sc=False)
              if D % 128 != 0 else None)

    npass = 2 if D > 64 else 1
    nstg = NCH // npass
    nbuf = 2 if D > 64 else 4

    @functools.partial(
        pl.kernel,
        out_type=jax.ShapeDtypeStruct((2 * NPAD, D), jnp.float32),
        mesh=_mesh,
        compiler_params=params,
        scratch_types=(
            [pltpu.VMEM((nstg, CH), jnp.int32),
             pltpu.VMEM((nstg, CH), jnp.int32)]
            + [pltpu.VMEM((CH, D), jnp.float32) for _ in range(nbuf)]
            + [pltpu.VMEM_SHARED((NPAD, D), jnp.float32)]
            + [pltpu.SemaphoreType.DMA for _ in range(nbuf)]
        ),
    )
    def k(hs_hbm, src_hbm, dst_hbm, zeros_hbm, out_hbm,
          src_all, dst_all, *rest):
        bufs = rest[:nbuf]
        acc = rest[nbuf]
        sems = rest[nbuf + 1:]
        c = lax.axis_index("c")
        s = lax.axis_index("s")
        wid = s * 2 + c
        rbase = s * ROWS_PER_TILE

        @pl.when(c == 0)
        def _():
            pltpu.sync_copy(hs_hbm.at[pl.ds(rbase, ROWS_PER_TILE)],
                            acc.at[pl.ds(rbase, ROWS_PER_TILE)])

        @pl.when(c != 0)
        def _():
            pltpu.sync_copy(zeros_hbm, acc.at[pl.ds(rbase, ROWS_PER_TILE)])

        plsc.subcore_barrier()

        for half in range(npass):
            pltpu.sync_copy(
                src_hbm.at[pl.ds(wid * NCH + half * nstg, nstg)], src_all)
            pltpu.sync_copy(
                dst_hbm.at[pl.ds(wid * NCH + half * nstg, nstg)], dst_all)
            for t in range(nbuf):
                pltpu.async_copy(hs_hbm.at[src_all.at[t]], bufs[t], sems[t])

            def body(j, carry):
                base = nbuf * j
                for t in range(nbuf):
                    ck = base + t
                    pltpu.make_async_copy(
                        hs_hbm.at[src_all.at[ck]], bufs[t], sems[t]).wait()
                    pltpu.sync_copy(bufs[t], acc.at[dst_all.at[ck]], add=True)

                    @pl.when(ck + nbuf < nstg)
                    def _(t=t, ck=ck):
                        pltpu.async_copy(
                            hs_hbm.at[src_all.at[ck + nbuf]], bufs[t], sems[t])

                return carry

            lax.fori_loop(0, nstg // nbuf, body, 0)
        plsc.subcore_barrier()
        pltpu.sync_copy(acc.at[pl.ds(rbase, ROWS_PER_TILE)],
                        out_hbm.at[pl.ds(c * NPAD + rbase, ROWS_PER_TILE)])

    return k


_sc_scatter_hid = _make_sc_scatter(D_HID)
_sc_scatter_out = _make_sc_scatter(D_OUT)


BN = 2048


def _tc1_body(cnt_ref, x_ref, w_ref, dis_ref, hs_ref):
    cnt = cnt_ref[:, 0:1] + cnt_ref[:, 1:2]
    dis = lax.rsqrt(cnt + 1.0)
    dis_ref[...] = dis
    h = jnp.dot(x_ref[...], w_ref[...], preferred_element_type=jnp.float32)
    hs_ref[...] = h * dis


def _tc1(cnt2, x_p, W1):
    grid = NPAD // BN
    return pl.pallas_call(
        _tc1_body,
        grid=(grid,),
        in_specs=[
            pl.BlockSpec((BN, 2), lambda i: (i, 0)),
            pl.BlockSpec((BN, D_IN), lambda i: (i, 0)),
            pl.BlockSpec((D_IN, D_HID), lambda i: (0, 0)),
        ],
        out_specs=[
            pl.BlockSpec((BN, 1), lambda i: (i, 0)),
            pl.BlockSpec((BN, D_HID), lambda i: (i, 0)),
        ],
        out_shape=[
            jax.ShapeDtypeStruct((NPAD, 1), jnp.float32),
            jax.ShapeDtypeStruct((NPAD, D_HID), jnp.float32),
        ],
    )(cnt2, x_p, W1)


def _tc2_body(p0_ref, p1_ref, dis_ref, b_ref, w_ref, hs2_ref):
    agg = p0_ref[...] + p1_ref[...]
    dis = dis_ref[...]
    h1 = jnp.maximum(agg * dis + b_ref[...], 0.0)
    hs2_ref[...] = jnp.dot(h1, w_ref[...],
                           preferred_element_type=jnp.float32) * dis


def _tc2(p, dis, b1r, W2):
    grid = NPAD // BN
    nb = NPAD // BN
    return pl.pallas_call(
        _tc2_body,
        grid=(grid,),
        in_specs=[
            pl.BlockSpec((BN, D_HID), lambda i: (i, 0)),
            pl.BlockSpec((BN, D_HID), lambda i: (i + nb, 0)),
            pl.BlockSpec((BN, 1), lambda i: (i, 0)),
            pl.BlockSpec((1, D_HID), lambda i: (0, 0)),
            pl.BlockSpec((D_HID, D_OUT), lambda i: (0, 0)),
        ],
        out_specs=pl.BlockSpec((BN, D_OUT), lambda i: (i, 0)),
        out_shape=jax.ShapeDtypeStruct((NPAD, D_OUT), jnp.float32),
    )(p, p, dis, b1r, W2)


def _tc3_body(q0_ref, q1_ref, dis_ref, b_ref, out_ref):
    agg = q0_ref[...] + q1_ref[...]
    out_ref[...] = agg * dis_ref[...] + b_ref[...]


def _tc3(q, dis, b2r):
    grid = NPAD // BN
    nb = NPAD // BN
    return pl.pallas_call(
        _tc3_body,
        grid=(grid,),
        in_specs=[
            pl.BlockSpec((BN, D_OUT), lambda i: (i, 0)),
            pl.BlockSpec((BN, D_OUT), lambda i: (i + nb, 0)),
            pl.BlockSpec((BN, 1), lambda i: (i, 0)),
            pl.BlockSpec((1, D_OUT), lambda i: (0, 0)),
        ],
        out_specs=pl.BlockSpec((BN, D_OUT), lambda i: (i, 0)),
        out_shape=jax.ShapeDtypeStruct((N_NODES, D_OUT), jnp.float32),
    )(q, q, dis, b2r)


def kernel(x, edge_index, W1, b1, W2, b2):
    src2 = jnp.concatenate([edge_index[0], _PAD_SRC]).reshape(EPAD // CH, CH)
    dst2 = jnp.concatenate([edge_index[1], _PAD_DST]).reshape(EPAD // CH, CH)
    zeros1 = jnp.zeros((ROWS_PER_TILE,), jnp.float32)
    zeros_hid = jnp.zeros((ROWS_PER_TILE, D_HID), jnp.float32)
    zeros_out = jnp.zeros((ROWS_PER_TILE, D_OUT), jnp.float32)

    cnt = _sc_count(dst2, zeros1)
    cnt2 = cnt.reshape(2, NPAD).T
    dis, hs1 = _tc1(cnt2, x, W1)
    p = _sc_scatter_hid(hs1, src2, dst2, zeros_hid)
    hs2 = _tc2(p, dis, b1.reshape(1, D_HID), W2)
    q = _sc_scatter_out(hs2, src2, dst2, zeros_out)
    return _tc3(q, dis, b2.reshape(1, D_OUT))
